# Initial kernel scaffold; baseline (speedup 1.0000x reference)
import jax
import jax.numpy as jnp
from jax.experimental import pallas as pl


def kernel(x, edge_index, batch, W1_0, b1_0, W2_0, b2_0, W1_1, b1_1, W2_1, b2_1, W1_2, b1_2, W2_2, b2_2, ln_g, ln_b):
    raise NotImplementedError("write your pallas kernel here")


# speedup vs baseline: 4.5898x; 4.5898x over previous
import functools

import jax
import jax.numpy as jnp
from jax import lax
from jax.experimental import pallas as pl
from jax.experimental.pallas import tpu as pltpu
from jax.experimental.pallas import tpu_sc as plsc

N = 10000
E = 320000
D = 128
G = 64

NC = 2
NS = 16
NW = NC * NS
EPW = E // NW
CH = 80
NCHUNK = EPW // CH
N_PAD = 10240
RPT = N_PAD // NS


def _seg_sum_body(h_hbm, src_hbm, dst_hbm, zeros_hbm, out_hbm,
                  sidx, didx, rows, agg_sh, sem):
    c = lax.axis_index("c")
    s = lax.axis_index("s")
    wid = s * NC + c
    pltpu.sync_copy(zeros_hbm.at[pl.ds(s * RPT, RPT)],
                    agg_sh.at[pl.ds(s * RPT, RPT)])
    plsc.subcore_barrier()

    def body(i, carry):
        off = wid * EPW + i * CH
        pltpu.sync_copy(src_hbm.at[pl.ds(off, CH)], sidx)
        pltpu.sync_copy(dst_hbm.at[pl.ds(off, CH)], didx)
        pltpu.async_copy(h_hbm.at[sidx], rows, sem).wait()
        pltpu.sync_copy(rows, agg_sh.at[didx], add=True)
        return carry

    lax.fori_loop(0, NCHUNK, body, 0)
    plsc.subcore_barrier()
    pltpu.sync_copy(agg_sh.at[pl.ds(s * RPT, RPT)],
                    out_hbm.at[pl.ds(c * N_PAD + s * RPT, RPT)])


def _sc_segment_sum(h, src, dst, zeros):
    mesh = plsc.VectorSubcoreMesh(core_axis_name="c", subcore_axis_name="s",
                                  num_cores=NC, num_subcores=NS)
    f = pl.kernel(
        _seg_sum_body,
        out_type=jax.ShapeDtypeStruct((NC * N_PAD, D), jnp.float32),
        mesh=mesh,
        scratch_types=[
            pltpu.VMEM((CH,), jnp.int32),
            pltpu.VMEM((CH,), jnp.int32),
            pltpu.VMEM((CH, D), jnp.float32),
            pltpu.VMEM_SHARED((N_PAD, D), jnp.float32),
            pltpu.SemaphoreType.DMA,
        ],
    )
    return f(h, src, dst, zeros).reshape(NC, N_PAD, D)


def _mlp_body(h_ref, a0_ref, a1_ref, w1_ref, b1_ref, w2_ref, b2_ref, o_ref,
              *, relu_out):
    z = h_ref[...] + a0_ref[0] + a1_ref[0]
    z1 = jnp.dot(z, w1_ref[...], preferred_element_type=jnp.float32)
    z1 = jnp.maximum(z1 + b1_ref[...], 0.0)
    z2 = jnp.dot(z1, w2_ref[...], preferred_element_type=jnp.float32)
    z2 = z2 + b2_ref[...]
    if relu_out:
        z2 = jnp.maximum(z2, 0.0)
    o_ref[...] = z2


MLP_BLK = 1000
MLP_NB = N // MLP_BLK


def _tc_mlp(h, agg2, W1, b1, W2, b2, relu_out):
    grid = (MLP_NB,)
    return pl.pallas_call(
        functools.partial(_mlp_body, relu_out=relu_out),
        grid=grid,
        in_specs=[
            pl.BlockSpec((MLP_BLK, D), lambda i: (i, 0)),
            pl.BlockSpec((1, MLP_BLK, D), lambda i: (0, i, 0)),
            pl.BlockSpec((1, MLP_BLK, D), lambda i: (1, i, 0)),
            pl.BlockSpec((D, D), lambda i: (0, 0)),
            pl.BlockSpec((1, D), lambda i: (0, 0)),
            pl.BlockSpec((D, D), lambda i: (0, 0)),
            pl.BlockSpec((1, D), lambda i: (0, 0)),
        ],
        out_specs=pl.BlockSpec((MLP_BLK, D), lambda i: (i, 0)),
        out_shape=jax.ShapeDtypeStruct((N, D), jnp.float32),
    )(h, agg2, agg2, W1, b1, W2, b2)


def _pool_body(h_ref, batch_ref, g_ref, bb_ref, o_ref, acc_ref, cnt_ref):
    i = pl.program_id(0)

    @pl.when(i == 0)
    def _init():
        acc_ref[...] = jnp.zeros_like(acc_ref)
        cnt_ref[...] = jnp.zeros_like(cnt_ref)

    b = batch_ref[...]
    onehot = (b == lax.broadcasted_iota(jnp.int32, (1, G), 1)
              ).astype(jnp.float32)
    acc_ref[...] += lax.dot_general(
        onehot, h_ref[...], (((0,), (0,)), ((), ())),
        preferred_element_type=jnp.float32)
    cnt_ref[...] += lax.dot_general(
        onehot, jnp.ones_like(h_ref), (((0,), (0,)), ((), ())),
        preferred_element_type=jnp.float32)

    @pl.when(i == MLP_NB - 1)
    def _final():
        pooled = acc_ref[...] / jnp.maximum(cnt_ref[...], 1.0)
        mu = jnp.mean(pooled, axis=1, keepdims=True)
        var = jnp.mean((pooled - mu) ** 2, axis=1, keepdims=True)
        o_ref[...] = ((pooled - mu) * lax.rsqrt(var + 1e-5) * g_ref[...]
                      + bb_ref[...])


def _tc_pool_ln(h, batch2d, ln_g, ln_b):
    return pl.pallas_call(
        _pool_body,
        grid=(MLP_NB,),
        in_specs=[
            pl.BlockSpec((MLP_BLK, D), lambda i: (i, 0)),
            pl.BlockSpec((MLP_BLK, 1), lambda i: (i, 0)),
            pl.BlockSpec((1, D), lambda i: (0, 0)),
            pl.BlockSpec((1, D), lambda i: (0, 0)),
        ],
        out_specs=pl.BlockSpec((G, D), lambda i: (0, 0)),
        out_shape=jax.ShapeDtypeStruct((G, D), jnp.float32),
        scratch_shapes=[
            pltpu.VMEM((G, D), jnp.float32),
            pltpu.VMEM((G, D), jnp.float32),
        ],
    )(h, batch2d, ln_g, ln_b)


def kernel(x, edge_index, batch, W1_0, b1_0, W2_0, b2_0, W1_1, b1_1, W2_1,
           b2_1, W1_2, b1_2, W2_2, b2_2, ln_g, ln_b):
    src = edge_index[0]
    dst = edge_index[1]
    zeros = jnp.zeros((N_PAD, D), jnp.float32)
    batch2d = batch.reshape(N, 1)
    params = [
        (W1_0, b1_0.reshape(1, D), W2_0, b2_0.reshape(1, D)),
        (W1_1, b1_1.reshape(1, D), W2_1, b2_1.reshape(1, D)),
        (W1_2, b1_2.reshape(1, D), W2_2, b2_2.reshape(1, D)),
    ]
    h = x
    for li, (W1, b1, W2, b2) in enumerate(params):
        agg2 = _sc_segment_sum(h, src, dst, zeros)
        h = _tc_mlp(h, agg2, W1, b1, W2, b2, relu_out=(li < 2))
    return _tc_pool_ln(h, batch2d, ln_g.reshape(1, D), ln_b.reshape(1, D))
